# baseline (speedup 1.0000x reference)
import functools

import jax
import jax.numpy as jnp
from jax.experimental import pallas as pl
from jax.experimental.pallas import tpu as pltpu

_Q = 1024
_D = 512
_KG = 5000
_KC = 500
_NC = 5
_KGC = _KG // _NC

_DEF = jax.lax.Precision.DEFAULT
_NEG = -1e30


def _sim(q, p):
    return jax.lax.dot_general(
        q, p, (((1,), (1,)), ((), ())), preferred_element_type=jnp.float32,
        precision=_DEF)


def _wsum(e, p):
    return jax.lax.dot_general(
        e, p, (((1,), (0,)), ((), ())), preferred_element_type=jnp.float32,
        precision=_DEF)


def _retrieve(q, p):
    sim = _sim(q, p)
    m = jnp.max(sim, axis=-1, keepdims=True)
    e = jnp.exp(sim - m)
    s = jnp.sum(e, axis=-1, keepdims=True)
    return _wsum(e, p) * (1.0 / s)


def _body(q_ref, gc_ref, pa_ref, pb_ref, w1_ref, b1_ref, w2t_ref, b2_ref,
          o_ref, acc_ref, m_ref, s_ref, cr_ref):
    i = pl.program_id(0)
    q = q_ref[...]

    @pl.when(i == 0)
    def _init():
        ra = _retrieve(q, pa_ref[...])
        rb = _retrieve(q, pb_ref[...])
        cr_ref[...] = 0.5 * (ra + rb)
        m_ref[...] = jnp.full((_Q, 1), _NEG, jnp.float32)
        s_ref[...] = jnp.zeros((_Q, 1), jnp.float32)
        acc_ref[...] = jnp.zeros((_Q, _D), jnp.float32)

    gc = gc_ref[...]
    sim = _sim(q, gc)
    m_old = m_ref[...]
    m_new = jnp.maximum(m_old, jnp.max(sim, axis=-1, keepdims=True))
    scale = jnp.exp(m_old - m_new)
    e = jnp.exp(sim - m_new)
    s_ref[...] = s_ref[...] * scale + jnp.sum(e, axis=-1, keepdims=True)
    acc_ref[...] = acc_ref[...] * scale + _wsum(e, gc)
    m_ref[...] = m_new

    @pl.when(i == _NC - 1)
    def _finish():
        rg = acc_ref[...] * (1.0 / s_ref[...])
        cr = cr_ref[...]
        comb = jnp.concatenate([cr, rg], axis=-1)
        h = jax.lax.dot_general(
            comb, w1_ref[...], (((1,), (0,)), ((), ())),
            preferred_element_type=jnp.float32, precision=_DEF) + b1_ref[...]
        h = 0.5 * h * (1.0 + jax.lax.erf(h * 0.7071067811865476))
        logit = jnp.sum(h * w2t_ref[...], axis=-1, keepdims=True) + b2_ref[...]
        gate = jax.nn.sigmoid(logit)
        o_ref[...] = gate * cr + (1.0 - gate) * rg


@functools.partial(jax.jit, static_argnames=())
def kernel(query, global_patterns, classA_patterns, classB_patterns,
           W1, b1, W2, b2):
    out = pl.pallas_call(
        _body,
        grid=(_NC,),
        in_specs=[
            pl.BlockSpec((_Q, _D), lambda i: (0, 0)),
            pl.BlockSpec((_KGC, _D), lambda i: (i, 0)),
            pl.BlockSpec((_KC, _D), lambda i: (0, 0)),
            pl.BlockSpec((_KC, _D), lambda i: (0, 0)),
            pl.BlockSpec((2 * _D, 64), lambda i: (0, 0)),
            pl.BlockSpec((1, 64), lambda i: (0, 0)),
            pl.BlockSpec((1, 64), lambda i: (0, 0)),
            pl.BlockSpec((1, 1), lambda i: (0, 0)),
        ],
        out_specs=pl.BlockSpec((_Q, _D), lambda i: (0, 0)),
        out_shape=jax.ShapeDtypeStruct((_Q, _D), jnp.float32),
        scratch_shapes=[
            pltpu.VMEM((_Q, _D), jnp.float32),
            pltpu.VMEM((_Q, 1), jnp.float32),
            pltpu.VMEM((_Q, 1), jnp.float32),
            pltpu.VMEM((_Q, _D), jnp.float32),
        ],
    )(query, global_patterns, classA_patterns, classB_patterns,
      W1, b1.reshape(1, 64), W2.reshape(1, 64), b2.reshape(1, 1))
    return out


# speedup vs baseline: 1.0010x; 1.0010x over previous
import functools

import jax
import jax.numpy as jnp
from jax.experimental import pallas as pl
from jax.experimental.pallas import tpu as pltpu

_Q = 1024
_D = 512
_KG = 5000
_KC = 500
_NC = 5
_KGC = _KG // _NC

_DEF = jax.lax.Precision.DEFAULT


def _sim(q, p):
    return jax.lax.dot_general(
        q, p, (((1,), (1,)), ((), ())), preferred_element_type=jnp.float32,
        precision=_DEF)


def _wsum(e, p):
    return jax.lax.dot_general(
        e, p, (((1,), (0,)), ((), ())), preferred_element_type=jnp.float32,
        precision=_DEF)


def _retrieve(q, p):
    sim = _sim(q, p)
    m = jnp.max(sim, axis=-1, keepdims=True)
    e = jnp.exp(sim - m)
    s = jnp.sum(e, axis=-1, keepdims=True)
    return _wsum(e, p) * (1.0 / s)


def _body(q_ref, gc_ref, pa_ref, pb_ref, w1_ref, b1_ref, w2t_ref, b2_ref,
          o_ref, num_ref, m_ref, s_ref, cr_ref):
    i = pl.program_id(0)
    q = q_ref[...]

    @pl.when(i == 0)
    def _classes():
        ra = _retrieve(q, pa_ref[...])
        rb = _retrieve(q, pb_ref[...])
        cr_ref[...] = 0.5 * (ra + rb)

    gc = gc_ref[...]
    sim = _sim(q, gc)
    m = jnp.max(sim, axis=-1, keepdims=True)
    e = jnp.exp(sim - m)
    s_ref[i] = jnp.sum(e, axis=-1, keepdims=True)
    num_ref[i] = _wsum(e, gc)
    m_ref[i] = m

    @pl.when(i == _NC - 1)
    def _finish():
        mg = m_ref[0]
        for j in range(1, _NC):
            mg = jnp.maximum(mg, m_ref[j])
        num = jnp.zeros((_Q, _D), jnp.float32)
        s = jnp.zeros((_Q, 1), jnp.float32)
        for j in range(_NC):
            w = jnp.exp(m_ref[j] - mg)
            num = num + num_ref[j] * w
            s = s + s_ref[j] * w
        rg = num * (1.0 / s)

        cr = cr_ref[...]
        comb = jnp.concatenate([cr, rg], axis=-1)
        h = jax.lax.dot_general(
            comb, w1_ref[...], (((1,), (0,)), ((), ())),
            preferred_element_type=jnp.float32, precision=_DEF) + b1_ref[...]
        h = 0.5 * h * (1.0 + jax.lax.erf(h * 0.7071067811865476))
        logit = jnp.sum(h * w2t_ref[...], axis=-1, keepdims=True) + b2_ref[...]
        gate = jax.nn.sigmoid(logit)
        o_ref[...] = gate * cr + (1.0 - gate) * rg


@functools.partial(jax.jit, static_argnames=())
def kernel(query, global_patterns, classA_patterns, classB_patterns,
           W1, b1, W2, b2):
    out = pl.pallas_call(
        _body,
        grid=(_NC,),
        in_specs=[
            pl.BlockSpec((_Q, _D), lambda i: (0, 0)),
            pl.BlockSpec((_KGC, _D), lambda i: (i, 0)),
            pl.BlockSpec((_KC, _D), lambda i: (0, 0)),
            pl.BlockSpec((_KC, _D), lambda i: (0, 0)),
            pl.BlockSpec((2 * _D, 64), lambda i: (0, 0)),
            pl.BlockSpec((1, 64), lambda i: (0, 0)),
            pl.BlockSpec((1, 64), lambda i: (0, 0)),
            pl.BlockSpec((1, 1), lambda i: (0, 0)),
        ],
        out_specs=pl.BlockSpec((_Q, _D), lambda i: (0, 0)),
        out_shape=jax.ShapeDtypeStruct((_Q, _D), jnp.float32),
        scratch_shapes=[
            pltpu.VMEM((_NC, _Q, _D), jnp.float32),
            pltpu.VMEM((_NC, _Q, 1), jnp.float32),
            pltpu.VMEM((_NC, _Q, 1), jnp.float32),
            pltpu.VMEM((_Q, _D), jnp.float32),
        ],
    )(query, global_patterns, classA_patterns, classB_patterns,
      W1, b1.reshape(1, 64), W2.reshape(1, 64), b2.reshape(1, 1))
    return out


# speedup vs baseline: 1.0513x; 1.0502x over previous
import functools

import jax
import jax.numpy as jnp
from jax.experimental import pallas as pl

_Q = 1024
_D = 512
_BQ = 1024
_DEF = jax.lax.Precision.DEFAULT


def _retrieve(qb, p):
    sim = jax.lax.dot_general(
        qb, p, (((1,), (1,)), ((), ())), preferred_element_type=jnp.float32,
        precision=_DEF)
    m = jnp.max(sim, axis=-1, keepdims=True)
    e = jnp.exp(sim - m)
    s = jnp.sum(e, axis=-1, keepdims=True)
    num = jax.lax.dot_general(
        e.astype(jnp.bfloat16), p, (((1,), (0,)), ((), ())),
        preferred_element_type=jnp.float32, precision=_DEF)
    return num * (1.0 / s)


def _body(qb_ref, pg_ref, pa_ref, pb_ref, w1_ref, b1_ref, w2t_ref, b2_ref,
          o_ref):
    qb = qb_ref[...].astype(jnp.bfloat16)
    rg = _retrieve(qb, pg_ref[...].astype(jnp.bfloat16))
    ra = _retrieve(qb, pa_ref[...].astype(jnp.bfloat16))
    rb = _retrieve(qb, pb_ref[...].astype(jnp.bfloat16))
    cr = 0.5 * (ra + rb)

    comb = jnp.concatenate([cr, rg], axis=-1)
    h = jax.lax.dot_general(
        comb.astype(jnp.bfloat16), w1_ref[...].astype(jnp.bfloat16),
        (((1,), (0,)), ((), ())),
        preferred_element_type=jnp.float32, precision=_DEF) + b1_ref[...]
    h = 0.5 * h * (1.0 + jax.lax.erf(h * 0.7071067811865476))
    logit = jnp.sum(h * w2t_ref[...], axis=-1, keepdims=True) + b2_ref[...]
    gate = jax.nn.sigmoid(logit)
    o_ref[...] = gate * cr + (1.0 - gate) * rg


@functools.partial(jax.jit, static_argnames=())
def kernel(query, global_patterns, classA_patterns, classB_patterns,
           W1, b1, W2, b2):
    kg = global_patterns.shape[0]
    kc = classA_patterns.shape[0]
    grid = (_Q // _BQ,)
    out = pl.pallas_call(
        _body,
        grid=grid,
        in_specs=[
            pl.BlockSpec((_BQ, _D), lambda i: (i, 0)),
            pl.BlockSpec((kg, _D), lambda i: (0, 0)),
            pl.BlockSpec((kc, _D), lambda i: (0, 0)),
            pl.BlockSpec((kc, _D), lambda i: (0, 0)),
            pl.BlockSpec((2 * _D, 64), lambda i: (0, 0)),
            pl.BlockSpec((1, 64), lambda i: (0, 0)),
            pl.BlockSpec((1, 64), lambda i: (0, 0)),
            pl.BlockSpec((1, 1), lambda i: (0, 0)),
        ],
        out_specs=pl.BlockSpec((_BQ, _D), lambda i: (i, 0)),
        out_shape=jax.ShapeDtypeStruct((_Q, _D), jnp.float32),
    )(query, global_patterns, classA_patterns, classB_patterns,
      W1, b1.reshape(1, 64), W2.reshape(1, 64), b2.reshape(1, 1))
    return out
